# baseline (speedup 1.0000x reference)
import functools

import numpy as np

import jax
import jax.numpy as jnp
from jax import lax
from jax.experimental import pallas as pl
from jax.experimental.pallas import tpu as pltpu
from jax.experimental.pallas import tpu_sc as plsc

N = 10000
E = 320000
G = 64
D_LIN = 64
D_CONV = 64
D_EDGE = 16
D_H = 128
D_HP = D_H // 2
D_OUT = 32

NC = 2
NS = 16
NW = NC * NS
L = 16

K = 128
NCHUNK = E // K
JMAX = -(-NCHUNK // NW)

_MPERM = np.concatenate(
    [np.concatenate([32 * gb + 2 * np.arange(16),
                     32 * gb + 2 * np.arange(16) + 1])
     for gb in range(4)])


def _round_bf16_bits(x):
    u = lax.bitcast_convert_type(x, jnp.uint32)
    return (u + jnp.uint32(0x7FFF) + ((u >> 16) & jnp.uint32(1))) >> 16


def _pack_pair(lo, hi):
    w = (_round_bf16_bits(hi) << 16) | _round_bf16_bits(lo)
    return lax.bitcast_convert_type(w, jnp.float32)


def _tables_body(lin_ref, conv_ref, wl_ref, wc_ref, bn_ref, wse_ref, wso_ref,
                 wde_ref, wdo_ref, a_ref, b_ref):
    x = (jnp.dot(lin_ref[...], wl_ref[...], preferred_element_type=jnp.float32)
         + jnp.dot(conv_ref[...], wc_ref[...], preferred_element_type=jnp.float32)
         + bn_ref[...])
    nodes = jnp.maximum(x, 0.0)
    a_ref[...] = _pack_pair(
        jnp.dot(nodes, wse_ref[...], preferred_element_type=jnp.float32),
        jnp.dot(nodes, wso_ref[...], preferred_element_type=jnp.float32))
    b_ref[...] = _pack_pair(
        jnp.dot(nodes, wde_ref[...], preferred_element_type=jnp.float32),
        jnp.dot(nodes, wdo_ref[...], preferred_element_type=jnp.float32))


def _node_tables(lin, conv, W_l, W_c, b_node, W_se, W_so, W_de, W_do):
    blk = 1000
    grid = N // blk
    return pl.pallas_call(
        _tables_body,
        grid=(grid,),
        in_specs=[
            pl.BlockSpec((blk, D_LIN), lambda i: (i, 0)),
            pl.BlockSpec((blk, D_CONV), lambda i: (i, 0)),
            pl.BlockSpec((D_LIN, D_H), lambda i: (0, 0)),
            pl.BlockSpec((D_CONV, D_H), lambda i: (0, 0)),
            pl.BlockSpec((1, D_H), lambda i: (0, 0)),
            pl.BlockSpec((D_H, D_HP), lambda i: (0, 0)),
            pl.BlockSpec((D_H, D_HP), lambda i: (0, 0)),
            pl.BlockSpec((D_H, D_HP), lambda i: (0, 0)),
            pl.BlockSpec((D_H, D_HP), lambda i: (0, 0)),
        ],
        out_specs=[
            pl.BlockSpec((blk, D_HP), lambda i: (i, 0)),
            pl.BlockSpec((blk, D_HP), lambda i: (i, 0)),
        ],
        out_shape=[
            jax.ShapeDtypeStruct((N, D_HP), jnp.float32),
            jax.ShapeDtypeStruct((N, D_HP), jnp.float32),
        ],
    )(lin, conv, W_l, W_c, b_node, W_se, W_so, W_de, W_do)


def _edge_term_body(ea_ref, w1_ref, b1_ref, w2e_ref, w2o_ref, bme_ref,
                    bmo_ref, c_ref):
    t = jnp.maximum(
        jnp.dot(ea_ref[...], w1_ref[...], preferred_element_type=jnp.float32)
        + b1_ref[...], 0.0)
    c_ref[...] = _pack_pair(
        jnp.dot(t, w2e_ref[...], preferred_element_type=jnp.float32)
        + bme_ref[...],
        jnp.dot(t, w2o_ref[...], preferred_element_type=jnp.float32)
        + bmo_ref[...])


def _edge_term(edge_attr, W1, b1, W2e, W2o, bme, bmo):
    blk = 5000
    grid = E // blk
    return pl.pallas_call(
        _edge_term_body,
        grid=(grid,),
        in_specs=[
            pl.BlockSpec((blk, D_EDGE), lambda i: (i, 0)),
            pl.BlockSpec((D_EDGE, D_H), lambda i: (0, 0)),
            pl.BlockSpec((1, D_H), lambda i: (0, 0)),
            pl.BlockSpec((D_H, D_HP), lambda i: (0, 0)),
            pl.BlockSpec((D_H, D_HP), lambda i: (0, 0)),
            pl.BlockSpec((1, D_HP), lambda i: (0, 0)),
            pl.BlockSpec((1, D_HP), lambda i: (0, 0)),
        ],
        out_specs=pl.BlockSpec((blk, D_HP), lambda i: (i, 0)),
        out_shape=jax.ShapeDtypeStruct((E, D_HP), jnp.float32),
    )(edge_attr, W1, b1, W2e, W2o, bme, bmo)


def _final_body(p_ref, wg_ref, bg_ref, out_ref):
    g = p_ref[0] + p_ref[1]
    out_ref[...] = (jnp.dot(g, wg_ref[...], preferred_element_type=jnp.float32)
                    + bg_ref[...])


def _final(partials, W_g, b_g):
    return pl.pallas_call(
        _final_body,
        out_shape=jax.ShapeDtypeStruct((G, D_OUT), jnp.float32),
    )(partials, W_g, b_g)


def _sc_body(a_hbm, b_hbm, c_hbm, src_hbm, dst_hbm, batch_hbm, out_hbm,
             srcv, dstv, egv, av, bv, cv, mv, acc_sh, batch_sh,
             sema, semb, semc, semg, semi, sems):
    cid = lax.axis_index("c")
    sid = lax.axis_index("s")
    wid = sid * NC + cid

    @pl.when(sid == 0)
    def _zero():
        def zrow(i, _):
            for gblk in range(D_H // L):
                mv[0, i, pl.ds(gblk * L, L)] = jnp.zeros((L,), jnp.float32)
            return 0
        lax.fori_loop(0, G, zrow, 0)
        pltpu.sync_copy(mv.at[0, pl.ds(0, G), :], acc_sh)
        pltpu.sync_copy(batch_hbm, batch_sh)

    plsc.subcore_barrier()

    def cidx_of(j):
        return j * NW + wid

    def issue_idx(j, q):
        off = cidx_of(j) * K
        pltpu.async_copy(src_hbm.at[pl.ds(off, K)], srcv.at[q], semi.at[q])
        pltpu.async_copy(dst_hbm.at[pl.ds(off, K)], dstv.at[q], semi.at[q])

    def wait_idx(q):
        pltpu.make_async_copy(src_hbm.at[pl.ds(0, K)], srcv.at[q],
                              semi.at[q]).wait()
        pltpu.make_async_copy(dst_hbm.at[pl.ds(0, K)], dstv.at[q],
                              semi.at[q]).wait()

    def issue_gathers(j, q):
        off = cidx_of(j) * K
        pltpu.async_copy(a_hbm.at[srcv.at[q]], av.at[q], sema.at[q])
        pltpu.async_copy(b_hbm.at[dstv.at[q]], bv.at[q], semb.at[q])
        pltpu.async_copy(c_hbm.at[pl.ds(off, K), :], cv.at[q], semc.at[q])
        pltpu.async_copy(batch_sh.at[srcv.at[q]], egv.at[q], semg.at[q])

    off0 = cidx_of(0) * K
    pltpu.sync_copy(src_hbm.at[pl.ds(off0, K)], srcv.at[0])
    pltpu.sync_copy(dst_hbm.at[pl.ds(off0, K)], dstv.at[0])
    issue_gathers(0, 0)

    @pl.when(cidx_of(1) < NCHUNK)
    def _pro1():
        issue_idx(1, 1)

    def half(j, p, q):
        @pl.when(cidx_of(j) < NCHUNK)
        def _do():
            @pl.when(j >= 1)
            def _drain():
                pltpu.make_async_copy(mv.at[q], acc_sh.at[egv.at[q]],
                                      sems.at[q]).wait()

            @pl.when(cidx_of(j + 1) < NCHUNK)
            def _pre():
                wait_idx(q)
                issue_gathers(j + 1, q)

            pltpu.make_async_copy(a_hbm.at[srcv.at[p]], av.at[p],
                                  sema.at[p]).wait()
            pltpu.make_async_copy(b_hbm.at[dstv.at[p]], bv.at[p],
                                  semb.at[p]).wait()
            pltpu.make_async_copy(c_hbm.at[pl.ds(0, K), :], cv.at[p],
                                  semc.at[p]).wait()
            pltpu.make_async_copy(batch_sh.at[srcv.at[p]], egv.at[p],
                                  semg.at[p]).wait()

            @pl.when(cidx_of(j + 2) < NCHUNK)
            def _preidx():
                issue_idx(j + 2, p)

            hi_mask = jnp.int32(-65536)

            def unpk(w):
                u = plsc.bitcast(w, jnp.int32)
                lo = plsc.bitcast(u << 16, jnp.float32)
                hi = plsc.bitcast(u & hi_mask, jnp.float32)
                return lo, hi

            def edge(e, _):
                for gblk in range(D_HP // L):
                    sl = pl.ds(gblk * L, L)
                    ae, ao = unpk(av[p, e, sl])
                    be, bo = unpk(bv[p, e, sl])
                    ce, co = unpk(cv[p, e, sl])
                    mv[p, e, pl.ds(gblk * 2 * L, L)] = jnp.maximum(
                        ae + be + ce, 0.0)
                    mv[p, e, pl.ds(gblk * 2 * L + L, L)] = jnp.maximum(
                        ao + bo + co, 0.0)
                return 0
            lax.fori_loop(0, K, edge, 0)

            pltpu.async_copy(mv.at[p], acc_sh.at[egv.at[p]], sems.at[p],
                             add=True)

    def chunk2(j2, _):
        half(j2 * 2, 0, 1)
        half(j2 * 2 + 1, 1, 0)
        return 0

    lax.fori_loop(0, (JMAX + 1) // 2, chunk2, 0)

    extra = NCHUNK - (JMAX - 1) * NW
    last = lax.select(wid < extra, (JMAX - 1) % 2, (JMAX - 2) % 2)
    pltpu.make_async_copy(mv.at[last], acc_sh.at[egv.at[last]],
                          sems.at[last]).wait()

    plsc.subcore_barrier()

    @pl.when(sid == 0)
    def _out():
        pltpu.sync_copy(acc_sh, out_hbm.at[cid])


_sc_segment = functools.partial(
    pl.kernel,
    out_type=jax.ShapeDtypeStruct((NC, G, D_H), jnp.float32),
    mesh=plsc.VectorSubcoreMesh(core_axis_name="c", subcore_axis_name="s"),
    compiler_params=pltpu.CompilerParams(needs_layout_passes=False,
                                         use_tc_tiling_on_sc=False),
    scratch_types=[
        pltpu.VMEM((2, K), jnp.int32),
        pltpu.VMEM((2, K), jnp.int32),
        pltpu.VMEM((2, K), jnp.int32),
        pltpu.VMEM((2, K, D_HP), jnp.float32),
        pltpu.VMEM((2, K, D_HP), jnp.float32),
        pltpu.VMEM((2, K, D_HP), jnp.float32),
        pltpu.VMEM((2, K, D_H), jnp.float32),
        pltpu.VMEM_SHARED((G, D_H), jnp.float32),
        pltpu.VMEM_SHARED((N,), jnp.int32),
        pltpu.SemaphoreType.DMA((2,)),
        pltpu.SemaphoreType.DMA((2,)),
        pltpu.SemaphoreType.DMA((2,)),
        pltpu.SemaphoreType.DMA((2,)),
        pltpu.SemaphoreType.DMA((2,)),
        pltpu.SemaphoreType.DMA((2,)),
    ],
)(_sc_body)


def kernel(node_linear_features, node_conv_features, edge_attr, edge_index,
           batch, num_graphs, W_node, b_node, W_edge_in, b_edge_in, W_em,
           b_em, W_g, b_g):
    del num_graphs
    W_l = W_node[:D_LIN]
    W_c = W_node[D_LIN:]
    W_s = W_em[:D_H]
    W_d = W_em[D_H:2 * D_H]
    W_e = W_em[2 * D_H:]
    a_tab, b_tab = _node_tables(node_linear_features, node_conv_features,
                                W_l, W_c, b_node.reshape(1, D_H),
                                W_s[:, 0::2], W_s[:, 1::2],
                                W_d[:, 0::2], W_d[:, 1::2])
    c_term = _edge_term(edge_attr, W_edge_in, b_edge_in.reshape(1, D_H),
                        W_e[:, 0::2], W_e[:, 1::2],
                        b_em[0::2].reshape(1, D_HP),
                        b_em[1::2].reshape(1, D_HP))
    src = edge_index[0]
    dst = edge_index[1]
    partials = _sc_segment(a_tab, b_tab, c_term, src, dst, batch)
    return _final(partials, W_g[_MPERM], b_g.reshape(1, D_OUT))


# speedup vs baseline: 1.6148x; 1.6148x over previous
import functools

import jax
import jax.numpy as jnp
from jax import lax
from jax.experimental import pallas as pl
from jax.experimental.pallas import tpu as pltpu
from jax.experimental.pallas import tpu_sc as plsc

N = 10000
E = 320000
G = 64
D_LIN = 64
D_CONV = 64
D_EDGE = 16
D_H = 128
D_OUT = 32

NC = 2
NS = 16
NW = NC * NS
L = 16

K = 128
NCHUNK = E // K
JMAX = -(-NCHUNK // NW)


def _tables_body(lin_ref, conv_ref, wl_ref, wc_ref, bn_ref, ws_ref, wd_ref,
                 a_ref, b_ref):
    x = (jnp.dot(lin_ref[...], wl_ref[...], preferred_element_type=jnp.float32)
         + jnp.dot(conv_ref[...], wc_ref[...], preferred_element_type=jnp.float32)
         + bn_ref[...])
    nodes = jnp.maximum(x, 0.0)
    a_ref[...] = jnp.dot(nodes, ws_ref[...], preferred_element_type=jnp.float32)
    b_ref[...] = jnp.dot(nodes, wd_ref[...], preferred_element_type=jnp.float32)


def _node_tables(lin, conv, W_l, W_c, b_node, W_s, W_d):
    blk = 1000
    grid = N // blk
    return pl.pallas_call(
        _tables_body,
        grid=(grid,),
        in_specs=[
            pl.BlockSpec((blk, D_LIN), lambda i: (i, 0)),
            pl.BlockSpec((blk, D_CONV), lambda i: (i, 0)),
            pl.BlockSpec((D_LIN, D_H), lambda i: (0, 0)),
            pl.BlockSpec((D_CONV, D_H), lambda i: (0, 0)),
            pl.BlockSpec((1, D_H), lambda i: (0, 0)),
            pl.BlockSpec((D_H, D_H), lambda i: (0, 0)),
            pl.BlockSpec((D_H, D_H), lambda i: (0, 0)),
        ],
        out_specs=[
            pl.BlockSpec((blk, D_H), lambda i: (i, 0)),
            pl.BlockSpec((blk, D_H), lambda i: (i, 0)),
        ],
        out_shape=[
            jax.ShapeDtypeStruct((N, D_H), jnp.float32),
            jax.ShapeDtypeStruct((N, D_H), jnp.float32),
        ],
    )(lin, conv, W_l, W_c, b_node, W_s, W_d)


def _edge_term_body(ea_ref, w1_ref, b1_ref, w2_ref, bm_ref, c_ref):
    t = jnp.maximum(
        jnp.dot(ea_ref[...], w1_ref[...], preferred_element_type=jnp.float32)
        + b1_ref[...], 0.0)
    c_ref[...] = (jnp.dot(t, w2_ref[...], preferred_element_type=jnp.float32)
                  + bm_ref[...])


def _edge_term(edge_attr, W1, b1, W2, b_em):
    blk = 5000
    grid = E // blk
    return pl.pallas_call(
        _edge_term_body,
        grid=(grid,),
        in_specs=[
            pl.BlockSpec((blk, D_EDGE), lambda i: (i, 0)),
            pl.BlockSpec((D_EDGE, D_H), lambda i: (0, 0)),
            pl.BlockSpec((1, D_H), lambda i: (0, 0)),
            pl.BlockSpec((D_H, D_H), lambda i: (0, 0)),
            pl.BlockSpec((1, D_H), lambda i: (0, 0)),
        ],
        out_specs=pl.BlockSpec((blk, D_H), lambda i: (i, 0)),
        out_shape=jax.ShapeDtypeStruct((E, D_H), jnp.float32),
    )(edge_attr, W1, b1, W2, b_em)


def _final_body(p_ref, wg_ref, bg_ref, out_ref):
    g = p_ref[0] + p_ref[1]
    out_ref[...] = (jnp.dot(g, wg_ref[...], preferred_element_type=jnp.float32)
                    + bg_ref[...])


def _final(partials, W_g, b_g):
    return pl.pallas_call(
        _final_body,
        out_shape=jax.ShapeDtypeStruct((G, D_OUT), jnp.float32),
    )(partials, W_g, b_g)


def _sc_body(a_hbm, b_hbm, c_hbm, src_hbm, dst_hbm, batch_hbm, out_hbm,
             srcv, dstv, egv, av, bv, cv, acc_sh,
             sema, semb, semc, semg, semi, sems):
    cid = lax.axis_index("c")
    sid = lax.axis_index("s")
    wid = sid * NC + cid

    @pl.when(sid == 0)
    def _zero():
        def zrow(i, _):
            for gblk in range(D_H // L):
                cv[0, i, pl.ds(gblk * L, L)] = jnp.zeros((L,), jnp.float32)
            return 0
        lax.fori_loop(0, G, zrow, 0)
        pltpu.sync_copy(cv.at[0, pl.ds(0, G), :], acc_sh)

    plsc.subcore_barrier()

    def cidx_of(j):
        return j * NW + wid

    def issue_idx(j, q):
        off = cidx_of(j) * K
        pltpu.async_copy(src_hbm.at[pl.ds(off, K)], srcv.at[q], semi.at[q])
        pltpu.async_copy(dst_hbm.at[pl.ds(off, K)], dstv.at[q], semi.at[q])

    def wait_idx(q):
        pltpu.make_async_copy(src_hbm.at[pl.ds(0, K)], srcv.at[q],
                              semi.at[q]).wait()
        pltpu.make_async_copy(dst_hbm.at[pl.ds(0, K)], dstv.at[q],
                              semi.at[q]).wait()

    def issue_gathers(j, q):
        off = cidx_of(j) * K
        pltpu.async_copy(a_hbm.at[srcv.at[q]], av.at[q], sema.at[q])
        pltpu.async_copy(b_hbm.at[dstv.at[q]], bv.at[q], semb.at[q])
        pltpu.async_copy(c_hbm.at[pl.ds(off, K), :], cv.at[q], semc.at[q])
        pltpu.async_copy(batch_hbm.at[srcv.at[q]], egv.at[q], semg.at[q])

    off0 = cidx_of(0) * K
    pltpu.sync_copy(src_hbm.at[pl.ds(off0, K)], srcv.at[0])
    pltpu.sync_copy(dst_hbm.at[pl.ds(off0, K)], dstv.at[0])
    issue_gathers(0, 0)

    @pl.when(cidx_of(1) < NCHUNK)
    def _pro1():
        issue_idx(1, 1)

    def half(j, p, q):
        @pl.when(cidx_of(j) < NCHUNK)
        def _do():
            @pl.when(j >= 1)
            def _drain():
                pltpu.make_async_copy(av.at[q], acc_sh.at[egv.at[q]],
                                      sems.at[q]).wait()

            @pl.when(cidx_of(j + 1) < NCHUNK)
            def _pre():
                wait_idx(q)
                issue_gathers(j + 1, q)

            pltpu.make_async_copy(a_hbm.at[srcv.at[p]], av.at[p],
                                  sema.at[p]).wait()
            pltpu.make_async_copy(b_hbm.at[dstv.at[p]], bv.at[p],
                                  semb.at[p]).wait()
            pltpu.make_async_copy(c_hbm.at[pl.ds(0, K), :], cv.at[p],
                                  semc.at[p]).wait()
            pltpu.make_async_copy(batch_hbm.at[srcv.at[p]], egv.at[p],
                                  semg.at[p]).wait()

            @pl.when(cidx_of(j + 2) < NCHUNK)
            def _preidx():
                issue_idx(j + 2, p)

            def edge(e, _):
                for gblk in range(D_H // L):
                    sl = pl.ds(gblk * L, L)
                    v = av[p, e, sl] + bv[p, e, sl] + cv[p, e, sl]
                    av[p, e, sl] = jnp.maximum(v, 0.0)
                return 0
            lax.fori_loop(0, K, edge, 0)

            pltpu.async_copy(av.at[p], acc_sh.at[egv.at[p]], sems.at[p],
                             add=True)

    def chunk2(j2, _):
        half(j2 * 2, 0, 1)
        half(j2 * 2 + 1, 1, 0)
        return 0

    lax.fori_loop(0, (JMAX + 1) // 2, chunk2, 0)

    extra = NCHUNK - (JMAX - 1) * NW
    last = lax.select(wid < extra, (JMAX - 1) % 2, (JMAX - 2) % 2)
    pltpu.make_async_copy(av.at[last], acc_sh.at[egv.at[last]],
                          sems.at[last]).wait()

    plsc.subcore_barrier()

    @pl.when(sid == 0)
    def _out():
        pltpu.sync_copy(acc_sh, out_hbm.at[cid])


_sc_segment = functools.partial(
    pl.kernel,
    out_type=jax.ShapeDtypeStruct((NC, G, D_H), jnp.float32),
    mesh=plsc.VectorSubcoreMesh(core_axis_name="c", subcore_axis_name="s"),
    scratch_types=[
        pltpu.VMEM((2, K), jnp.int32),
        pltpu.VMEM((2, K), jnp.int32),
        pltpu.VMEM((2, K), jnp.int32),
        pltpu.VMEM((2, K, D_H), jnp.float32),
        pltpu.VMEM((2, K, D_H), jnp.float32),
        pltpu.VMEM((2, K, D_H), jnp.float32),
        pltpu.VMEM_SHARED((G, D_H), jnp.float32),
        pltpu.SemaphoreType.DMA((2,)),
        pltpu.SemaphoreType.DMA((2,)),
        pltpu.SemaphoreType.DMA((2,)),
        pltpu.SemaphoreType.DMA((2,)),
        pltpu.SemaphoreType.DMA((2,)),
        pltpu.SemaphoreType.DMA((2,)),
    ],
)(_sc_body)


def kernel(node_linear_features, node_conv_features, edge_attr, edge_index,
           batch, num_graphs, W_node, b_node, W_edge_in, b_edge_in, W_em,
           b_em, W_g, b_g):
    del num_graphs
    W_l = W_node[:D_LIN]
    W_c = W_node[D_LIN:]
    W_s = W_em[:D_H]
    W_d = W_em[D_H:2 * D_H]
    W_e = W_em[2 * D_H:]
    a_tab, b_tab = _node_tables(node_linear_features, node_conv_features,
                                W_l, W_c, b_node.reshape(1, D_H), W_s, W_d)
    c_term = _edge_term(edge_attr, W_edge_in, b_edge_in.reshape(1, D_H),
                        W_e, b_em.reshape(1, D_H))
    src = edge_index[0]
    dst = edge_index[1]
    partials = _sc_segment(a_tab, b_tab, c_term, src, dst, batch)
    return _final(partials, W_g, b_g.reshape(1, D_OUT))


# speedup vs baseline: 1.7251x; 1.0683x over previous
import functools

import jax
import jax.numpy as jnp
from jax import lax
from jax.experimental import pallas as pl
from jax.experimental.pallas import tpu as pltpu
from jax.experimental.pallas import tpu_sc as plsc

N = 10000
E = 320000
G = 64
D_LIN = 64
D_CONV = 64
D_EDGE = 16
D_H = 128
D_OUT = 32

NC = 2
NS = 16
NW = NC * NS
L = 16

K = 128
EH = E // 2
NCHUNK = EH // K
JMAX = -(-NCHUNK // NW)


def _tables_body(lin_ref, conv_ref, wl_ref, wc_ref, bn_ref, ws_ref, wd_ref,
                 a_ref, b_ref):
    x = (jnp.dot(lin_ref[...], wl_ref[...], preferred_element_type=jnp.float32)
         + jnp.dot(conv_ref[...], wc_ref[...], preferred_element_type=jnp.float32)
         + bn_ref[...])
    nodes = jnp.maximum(x, 0.0)
    a_ref[...] = jnp.dot(nodes, ws_ref[...], preferred_element_type=jnp.float32)
    b_ref[...] = jnp.dot(nodes, wd_ref[...], preferred_element_type=jnp.float32)


def _node_tables(lin, conv, W_l, W_c, b_node, W_s, W_d):
    blk = 1000
    grid = N // blk
    return pl.pallas_call(
        _tables_body,
        grid=(grid,),
        in_specs=[
            pl.BlockSpec((blk, D_LIN), lambda i: (i, 0)),
            pl.BlockSpec((blk, D_CONV), lambda i: (i, 0)),
            pl.BlockSpec((D_LIN, D_H), lambda i: (0, 0)),
            pl.BlockSpec((D_CONV, D_H), lambda i: (0, 0)),
            pl.BlockSpec((1, D_H), lambda i: (0, 0)),
            pl.BlockSpec((D_H, D_H), lambda i: (0, 0)),
            pl.BlockSpec((D_H, D_H), lambda i: (0, 0)),
        ],
        out_specs=[
            pl.BlockSpec((blk, D_H), lambda i: (i, 0)),
            pl.BlockSpec((blk, D_H), lambda i: (i, 0)),
        ],
        out_shape=[
            jax.ShapeDtypeStruct((N, D_H), jnp.float32),
            jax.ShapeDtypeStruct((N, D_H), jnp.float32),
        ],
    )(lin, conv, W_l, W_c, b_node, W_s, W_d)


def _edge_term_body(ea_ref, w1_ref, b1_ref, w2_ref, bm_ref, c_ref):
    t = jnp.maximum(
        jnp.dot(ea_ref[...], w1_ref[...], preferred_element_type=jnp.float32)
        + b1_ref[...], 0.0)
    c_ref[...] = (jnp.dot(t, w2_ref[...], preferred_element_type=jnp.float32)
                  + bm_ref[...])


def _edge_term(edge_attr, W1, b1, W2, b_em):
    blk = 5000
    grid = edge_attr.shape[0] // blk
    return pl.pallas_call(
        _edge_term_body,
        grid=(grid,),
        in_specs=[
            pl.BlockSpec((blk, D_EDGE), lambda i: (i, 0)),
            pl.BlockSpec((D_EDGE, D_H), lambda i: (0, 0)),
            pl.BlockSpec((1, D_H), lambda i: (0, 0)),
            pl.BlockSpec((D_H, D_H), lambda i: (0, 0)),
            pl.BlockSpec((1, D_H), lambda i: (0, 0)),
        ],
        out_specs=pl.BlockSpec((blk, D_H), lambda i: (i, 0)),
        out_shape=jax.ShapeDtypeStruct((edge_attr.shape[0], D_H), jnp.float32),
    )(edge_attr, W1, b1, W2, b_em)


def _final_body(p1_ref, p2_ref, wg_ref, bg_ref, out_ref):
    g = p1_ref[0] + p1_ref[1] + p2_ref[0] + p2_ref[1]
    out_ref[...] = (jnp.dot(g, wg_ref[...], preferred_element_type=jnp.float32)
                    + bg_ref[...])


def _final(p1, p2, W_g, b_g):
    return pl.pallas_call(
        _final_body,
        out_shape=jax.ShapeDtypeStruct((G, D_OUT), jnp.float32),
    )(p1, p2, W_g, b_g)


def _sc_body(ebase, a_hbm, b_hbm, c_hbm, ei_hbm, batch_hbm, out_hbm,
             srcv, dstv, egv, av, bv, cv, acc_sh,
             sema, semb, semc, semg, semi, sems):
    cid = lax.axis_index("c")
    sid = lax.axis_index("s")
    wid = sid * NC + cid

    @pl.when(sid == 0)
    def _zero():
        def zrow(i, _):
            for gblk in range(D_H // L):
                cv[0, i, pl.ds(gblk * L, L)] = jnp.zeros((L,), jnp.float32)
            return 0
        lax.fori_loop(0, G, zrow, 0)
        pltpu.sync_copy(cv.at[0, pl.ds(0, G), :], acc_sh)

    plsc.subcore_barrier()

    def cidx_of(j):
        return j * NW + wid

    def issue_idx(j, q):
        off = ebase + cidx_of(j) * K
        pltpu.async_copy(ei_hbm.at[0, pl.ds(off, K)], srcv.at[q], semi.at[q])
        pltpu.async_copy(ei_hbm.at[1, pl.ds(off, K)], dstv.at[q], semi.at[q])

    def wait_idx(q):
        pltpu.make_async_copy(ei_hbm.at[0, pl.ds(0, K)], srcv.at[q],
                              semi.at[q]).wait()
        pltpu.make_async_copy(ei_hbm.at[1, pl.ds(0, K)], dstv.at[q],
                              semi.at[q]).wait()

    def issue_gathers(j, q):
        off = cidx_of(j) * K
        pltpu.async_copy(a_hbm.at[srcv.at[q]], av.at[q], sema.at[q])
        pltpu.async_copy(b_hbm.at[dstv.at[q]], bv.at[q], semb.at[q])
        pltpu.async_copy(c_hbm.at[pl.ds(off, K), :], cv.at[q], semc.at[q])
        pltpu.async_copy(batch_hbm.at[srcv.at[q]], egv.at[q], semg.at[q])

    off0 = ebase + cidx_of(0) * K
    pltpu.sync_copy(ei_hbm.at[0, pl.ds(off0, K)], srcv.at[0])
    pltpu.sync_copy(ei_hbm.at[1, pl.ds(off0, K)], dstv.at[0])
    issue_gathers(0, 0)

    @pl.when(cidx_of(1) < NCHUNK)
    def _pro1():
        issue_idx(1, 1)

    def half(j, p, q):
        @pl.when(cidx_of(j) < NCHUNK)
        def _do():
            @pl.when(j >= 1)
            def _drain():
                pltpu.make_async_copy(av.at[q], acc_sh.at[egv.at[q]],
                                      sems.at[q]).wait()

            @pl.when(cidx_of(j + 1) < NCHUNK)
            def _pre():
                wait_idx(q)
                issue_gathers(j + 1, q)

            pltpu.make_async_copy(a_hbm.at[srcv.at[p]], av.at[p],
                                  sema.at[p]).wait()
            pltpu.make_async_copy(b_hbm.at[dstv.at[p]], bv.at[p],
                                  semb.at[p]).wait()
            pltpu.make_async_copy(c_hbm.at[pl.ds(0, K), :], cv.at[p],
                                  semc.at[p]).wait()
            pltpu.make_async_copy(batch_hbm.at[srcv.at[p]], egv.at[p],
                                  semg.at[p]).wait()

            @pl.when(cidx_of(j + 2) < NCHUNK)
            def _preidx():
                issue_idx(j + 2, p)

            def edge(e, _):
                for gblk in range(D_H // L):
                    sl = pl.ds(gblk * L, L)
                    v = av[p, e, sl] + bv[p, e, sl] + cv[p, e, sl]
                    av[p, e, sl] = jnp.maximum(v, 0.0)
                return 0
            lax.fori_loop(0, K, edge, 0)

            pltpu.async_copy(av.at[p], acc_sh.at[egv.at[p]], sems.at[p],
                             add=True)

    def chunk2(j2, _):
        half(j2 * 2, 0, 1)
        half(j2 * 2 + 1, 1, 0)
        return 0

    lax.fori_loop(0, (JMAX + 1) // 2, chunk2, 0)

    extra = NCHUNK - (JMAX - 1) * NW
    last = lax.select(wid < extra, (JMAX - 1) % 2, (JMAX - 2) % 2)
    pltpu.make_async_copy(av.at[last], acc_sh.at[egv.at[last]],
                          sems.at[last]).wait()

    plsc.subcore_barrier()

    @pl.when(sid == 0)
    def _out():
        pltpu.sync_copy(acc_sh, out_hbm.at[cid])


def _sc_segment(ebase, a_tab, b_tab, c_term, edge_index, batch):
    body = functools.partial(_sc_body, ebase)
    return pl.kernel(
        body,
        out_type=jax.ShapeDtypeStruct((NC, G, D_H), jnp.float32),
        mesh=plsc.VectorSubcoreMesh(core_axis_name="c", subcore_axis_name="s"),
        scratch_types=[
        pltpu.VMEM((2, K), jnp.int32),
        pltpu.VMEM((2, K), jnp.int32),
        pltpu.VMEM((2, K), jnp.int32),
        pltpu.VMEM((2, K, D_H), jnp.float32),
        pltpu.VMEM((2, K, D_H), jnp.float32),
        pltpu.VMEM((2, K, D_H), jnp.float32),
        pltpu.VMEM_SHARED((G, D_H), jnp.float32),
        pltpu.SemaphoreType.DMA((2,)),
        pltpu.SemaphoreType.DMA((2,)),
        pltpu.SemaphoreType.DMA((2,)),
        pltpu.SemaphoreType.DMA((2,)),
        pltpu.SemaphoreType.DMA((2,)),
        pltpu.SemaphoreType.DMA((2,)),
        ],
    )(a_tab, b_tab, c_term, edge_index, batch)


def kernel(node_linear_features, node_conv_features, edge_attr, edge_index,
           batch, num_graphs, W_node, b_node, W_edge_in, b_edge_in, W_em,
           b_em, W_g, b_g):
    del num_graphs
    W_l = W_node[:D_LIN]
    W_c = W_node[D_LIN:]
    W_s = W_em[:D_H]
    W_d = W_em[D_H:2 * D_H]
    W_e = W_em[2 * D_H:]
    a_tab, b_tab = _node_tables(node_linear_features, node_conv_features,
                                W_l, W_c, b_node.reshape(1, D_H), W_s, W_d)
    b_in = b_edge_in.reshape(1, D_H)
    b_m = b_em.reshape(1, D_H)
    c1 = _edge_term(edge_attr[:EH], W_edge_in, b_in, W_e, b_m)
    p1 = _sc_segment(0, a_tab, b_tab, c1, edge_index, batch)
    c2 = _edge_term(edge_attr[EH:], W_edge_in, b_in, W_e, b_m)
    p2 = _sc_segment(EH, a_tab, b_tab, c2, edge_index, batch)
    return _final(p1, p2, W_g, b_g.reshape(1, D_OUT))
